# baseline (speedup 1.0000x reference)
import functools

import jax
import jax.numpy as jnp
from jax import lax
from jax.experimental import pallas as pl
from jax.experimental.pallas import tpu as pltpu
from jax.experimental.pallas import tpu_sc as plsc

D_MODEL = 64
NC = 2
NS = 16
NW = NC * NS
CHUNK = 128


@functools.lru_cache(maxsize=None)
def _build(n_chunks, d_model):
    mesh = plsc.VectorSubcoreMesh(core_axis_name="c", subcore_axis_name="s")
    b_total = NW * n_chunks * CHUNK

    @functools.partial(
        pl.kernel,
        mesh=mesh,
        compiler_params=pltpu.CompilerParams(use_tc_tiling_on_sc=False),
        out_type=jax.ShapeDtypeStruct((b_total, d_model), jnp.float32),
        scratch_types=[
            pltpu.VMEM((n_chunks, CHUNK), jnp.int32),
            pltpu.VMEM((CHUNK, d_model), jnp.float32),
            pltpu.SemaphoreType.DMA,
        ],
    )
    def body(table_hbm, idx_hbm, out_hbm, idx_v, rows_v, sem):
        wid = lax.axis_index("s") * NC + lax.axis_index("c")
        pltpu.sync_copy(idx_hbm.at[wid], idx_v)
        base = wid * (n_chunks * CHUNK)

        def step(g, carry):
            pltpu.async_copy(table_hbm.at[idx_v.at[g]], rows_v, sem).wait()
            pltpu.sync_copy(rows_v, out_hbm.at[pl.ds(base + g * CHUNK, CHUNK)])
            return carry

        lax.fori_loop(0, n_chunks, step, 0)

    return body


def kernel(token_ids, embed_table):
    b0, s = token_ids.shape
    v, d = embed_table.shape
    b_flat = b0 * s
    assert b_flat % (NW * CHUNK) == 0
    n_chunks = b_flat // (NW * CHUNK)
    idx = token_ids.reshape(NW, n_chunks, CHUNK).astype(jnp.int32)
    out = _build(n_chunks, d)(embed_table, idx)
    return out.reshape(b0, s, d)


# speedup vs baseline: 1.1163x; 1.1163x over previous
import functools

import jax
import jax.numpy as jnp
from jax import lax
from jax.experimental import pallas as pl
from jax.experimental.pallas import tpu as pltpu
from jax.experimental.pallas import tpu_sc as plsc

D_MODEL = 64
NC = 2
NS = 16
NW = NC * NS
CHUNK = 128
K = 4
GROUP = CHUNK * K


@functools.lru_cache(maxsize=None)
def _build(n_groups, d_model):
    mesh = plsc.VectorSubcoreMesh(core_axis_name="c", subcore_axis_name="s")
    n_chunks = n_groups * K
    b_total = NW * n_groups * GROUP
    assert n_groups % 2 == 0 and n_groups >= 4

    @functools.partial(
        pl.kernel,
        mesh=mesh,
        compiler_params=pltpu.CompilerParams(use_tc_tiling_on_sc=False),
        out_type=jax.ShapeDtypeStruct((b_total, d_model), jnp.float32),
        scratch_types=[
            pltpu.VMEM((n_chunks, CHUNK), jnp.int32),
            pltpu.VMEM((GROUP, d_model), jnp.float32),
            pltpu.VMEM((GROUP, d_model), jnp.float32),
            pltpu.SemaphoreType.DMA,
            pltpu.SemaphoreType.DMA,
        ],
    )
    def body(table_hbm, idx_hbm, out_hbm, idx_v, rows0, rows1, sem0, sem1):
        wid = lax.axis_index("s") * NC + lax.axis_index("c")
        pltpu.sync_copy(idx_hbm.at[wid], idx_v)
        base = wid * (n_groups * GROUP)

        def fire(g, buf, sem):
            for k in range(K):
                pltpu.async_copy(
                    table_hbm.at[idx_v.at[g * K + k]],
                    buf.at[pl.ds(k * CHUNK, CHUNK)],
                    sem,
                )

        def drain_writeback(g, buf, sem):
            pltpu.make_async_copy(table_hbm.at[pl.ds(0, GROUP)], buf, sem).wait()
            pltpu.sync_copy(buf, out_hbm.at[pl.ds(base + g * GROUP, GROUP)])

        fire(0, rows0, sem0)

        def step(go, carry):
            g = 2 * go
            fire(g + 1, rows1, sem1)
            drain_writeback(g, rows0, sem0)
            fire(g + 2, rows0, sem0)
            drain_writeback(g + 1, rows1, sem1)
            return carry

        lax.fori_loop(0, n_groups // 2 - 1, step, 0)

        g_last = n_groups - 2
        fire(g_last + 1, rows1, sem1)
        drain_writeback(g_last, rows0, sem0)
        drain_writeback(g_last + 1, rows1, sem1)

    return body


def kernel(token_ids, embed_table):
    b0, s = token_ids.shape
    v, d = embed_table.shape
    b_flat = b0 * s
    assert b_flat % (NW * GROUP) == 0
    n_groups = b_flat // (NW * GROUP)
    idx = token_ids.reshape(NW, n_groups * K, CHUNK).astype(jnp.int32)
    out = _build(n_groups, d)(embed_table, idx)
    return out.reshape(b0, s, d)
